# Initial kernel scaffold; baseline (speedup 1.0000x reference)
import jax
import jax.numpy as jnp
from jax.experimental import pallas as pl


def kernel(x, edge_index, W1, b1, Wmu, bmu, Wls, bls):
    raise NotImplementedError("write your pallas kernel here")


# speedup vs baseline: 15.6533x; 15.6533x over previous
import jax
import jax.numpy as jnp
from jax import lax
from jax.experimental import pallas as pl
from jax.experimental.pallas import tpu as pltpu
from jax.experimental.pallas import tpu_sc as plsc

N = 10000
NPAD = 10240
E = 320000
D_IN = 128
D_H = 128
D_OUT = 64
DHF = D_H // 2

NC = 2
NS = 16
NW = NC * NS
EPW = E // NW
C = 80
K = EPW // C
RPT = NPAD // NS

_sc_mesh = plsc.VectorSubcoreMesh(core_axis_name="c", subcore_axis_name="s")


def _deg_body(dst_hbm, out_hbm, dst_v, ones_v, zbuf, acc):
    cid = lax.axis_index("c")
    sid = lax.axis_index("s")
    wid = sid * NC + cid

    one16 = jnp.ones((16,), jnp.float32)
    zero16 = jnp.zeros((16,), jnp.float32)

    @pl.loop(0, C)
    def _(i):
        ones_v[i, :] = one16

    @pl.loop(0, RPT)
    def _(i):
        zbuf[i, :] = zero16

    pltpu.sync_copy(dst_hbm.at[wid], dst_v)
    pltpu.sync_copy(zbuf, acc.at[pl.ds(sid * RPT, RPT)])
    plsc.subcore_barrier()

    @pl.loop(0, K)
    def _(j):
        pltpu.sync_copy(ones_v, acc.at[dst_v.at[j]], add=True)

    plsc.subcore_barrier()
    pltpu.sync_copy(acc.at[pl.ds(sid * RPT, RPT)],
                    out_hbm.at[cid, pl.ds(sid * RPT, RPT)])


_deg_call = pl.kernel(
    _deg_body,
    out_type=jax.ShapeDtypeStruct((NC, NPAD, 16), jnp.float32),
    mesh=_sc_mesh,
    scratch_types=[
        pltpu.VMEM((K, C), jnp.int32),
        pltpu.VMEM((C, 16), jnp.float32),
        pltpu.VMEM((RPT, 16), jnp.float32),
        pltpu.VMEM_SHARED((NPAD, 16), jnp.float32),
    ],
    compiler_params=pltpu.CompilerParams(use_tc_tiling_on_sc=False),
)


def _agg_body(y_hbm, src_hbm, dst_hbm, out_hbm, src_v, dst_v, rows_v, zbuf, acc):
    cid = lax.axis_index("c")
    sid = lax.axis_index("s")
    wid = sid * NC + cid

    zero16 = jnp.zeros((16,), jnp.float32)

    @pl.loop(0, RPT)
    def _(i):
        @pl.loop(0, DHF, step=16)
        def _(c0):
            zbuf[i, pl.ds(c0, 16)] = zero16

    pltpu.sync_copy(src_hbm.at[wid], src_v)
    pltpu.sync_copy(dst_hbm.at[wid], dst_v)

    for half in range(2):
        pltpu.sync_copy(zbuf, acc.at[pl.ds(sid * RPT, RPT)])
        plsc.subcore_barrier()

        y_half = y_hbm.at[half]

        @pl.loop(0, K)
        def _(j):
            pltpu.sync_copy(y_half.at[src_v.at[j]], rows_v)
            pltpu.sync_copy(rows_v, acc.at[dst_v.at[j]], add=True)

        plsc.subcore_barrier()
        pltpu.sync_copy(acc.at[pl.ds(sid * RPT, RPT)],
                        out_hbm.at[cid, half, pl.ds(sid * RPT, RPT)])


_agg_call = pl.kernel(
    _agg_body,
    out_type=jax.ShapeDtypeStruct((NC, 2, NPAD, DHF), jnp.float32),
    mesh=_sc_mesh,
    scratch_types=[
        pltpu.VMEM((K, C), jnp.int32),
        pltpu.VMEM((K, C), jnp.int32),
        pltpu.VMEM((C, DHF), jnp.float32),
        pltpu.VMEM((RPT, DHF), jnp.float32),
        pltpu.VMEM_SHARED((NPAD, DHF), jnp.float32),
    ],
    compiler_params=pltpu.CompilerParams(use_tc_tiling_on_sc=False),
)


BR = 1000


def _mm_body(x_ref, w_ref, o_ref):
    o_ref[...] = jnp.dot(x_ref[...], w_ref[...],
                         preferred_element_type=jnp.float32)


def _mm(x, w):
    n, d = x.shape
    return pl.pallas_call(
        _mm_body,
        grid=(n // BR,),
        in_specs=[
            pl.BlockSpec((BR, d), lambda i: (i, 0)),
            pl.BlockSpec((d, w.shape[1]), lambda i: (0, 0)),
        ],
        out_specs=pl.BlockSpec((BR, w.shape[1]), lambda i: (i, 0)),
        out_shape=jax.ShapeDtypeStruct((n, w.shape[1]), jnp.float32),
    )(x, w)


def _norm_body(d0_ref, d1_ref, xw_ref, y_ref, dinv_ref):
    deg = 1.0 + d0_ref[...] + d1_ref[...]
    dinv = lax.rsqrt(deg)
    dinv_ref[...] = dinv
    y = dinv * xw_ref[...]
    y_ref[0] = y[:, :DHF]
    y_ref[1] = y[:, DHF:]


def _norm(d0, d1, xw):
    return pl.pallas_call(
        _norm_body,
        grid=(N // BR,),
        in_specs=[
            pl.BlockSpec((BR, 1), lambda i: (i, 0)),
            pl.BlockSpec((BR, 1), lambda i: (i, 0)),
            pl.BlockSpec((BR, D_H), lambda i: (i, 0)),
        ],
        out_specs=[
            pl.BlockSpec((2, BR, DHF), lambda i: (0, i, 0)),
            pl.BlockSpec((BR, 1), lambda i: (i, 0)),
        ],
        out_shape=[
            jax.ShapeDtypeStruct((2, N, DHF), jnp.float32),
            jax.ShapeDtypeStruct((N, 1), jnp.float32),
        ],
    )(d0, d1, xw)


def _layer1_body(p_ref, xw_ref, dinv_ref, b_ref, h_ref, y2_ref):
    dinv = dinv_ref[...]
    agg = jnp.concatenate(
        [p_ref[0, 0] + p_ref[1, 0], p_ref[0, 1] + p_ref[1, 1]], axis=-1)
    pre = dinv * agg + (dinv * dinv) * xw_ref[...]
    h = jnp.maximum(pre + b_ref[...], 0.0)
    h_ref[...] = h
    y2 = dinv * h
    y2_ref[0] = y2[:, :DHF]
    y2_ref[1] = y2[:, DHF:]


def _layer1(p, xw, dinv, b1):
    return pl.pallas_call(
        _layer1_body,
        grid=(N // BR,),
        in_specs=[
            pl.BlockSpec((NC, 2, BR, DHF), lambda i: (0, 0, i, 0)),
            pl.BlockSpec((BR, D_H), lambda i: (i, 0)),
            pl.BlockSpec((BR, 1), lambda i: (i, 0)),
            pl.BlockSpec((1, D_H), lambda i: (0, 0)),
        ],
        out_specs=[
            pl.BlockSpec((BR, D_H), lambda i: (i, 0)),
            pl.BlockSpec((2, BR, DHF), lambda i: (0, i, 0)),
        ],
        out_shape=[
            jax.ShapeDtypeStruct((N, D_H), jnp.float32),
            jax.ShapeDtypeStruct((2, N, DHF), jnp.float32),
        ],
    )(p, xw, dinv, b1)


def _heads_body(q_ref, h_ref, dinv_ref, wmu_ref, bmu_ref, wls_ref, bls_ref,
                mu_ref, ls_ref):
    dinv = dinv_ref[...]
    agg = jnp.concatenate(
        [q_ref[0, 0] + q_ref[1, 0], q_ref[0, 1] + q_ref[1, 1]], axis=-1)
    ah = dinv * agg + (dinv * dinv) * h_ref[...]
    mu_ref[...] = jnp.dot(ah, wmu_ref[...],
                          preferred_element_type=jnp.float32) + bmu_ref[...]
    ls_ref[...] = jnp.dot(ah, wls_ref[...],
                          preferred_element_type=jnp.float32) + bls_ref[...]


def _heads(q, h, dinv, Wmu, bmu, Wls, bls):
    return pl.pallas_call(
        _heads_body,
        grid=(N // BR,),
        in_specs=[
            pl.BlockSpec((NC, 2, BR, DHF), lambda i: (0, 0, i, 0)),
            pl.BlockSpec((BR, D_H), lambda i: (i, 0)),
            pl.BlockSpec((BR, 1), lambda i: (i, 0)),
            pl.BlockSpec((D_H, D_OUT), lambda i: (0, 0)),
            pl.BlockSpec((1, D_OUT), lambda i: (0, 0)),
            pl.BlockSpec((D_H, D_OUT), lambda i: (0, 0)),
            pl.BlockSpec((1, D_OUT), lambda i: (0, 0)),
        ],
        out_specs=[
            pl.BlockSpec((BR, D_OUT), lambda i: (i, 0)),
            pl.BlockSpec((BR, D_OUT), lambda i: (i, 0)),
        ],
        out_shape=[
            jax.ShapeDtypeStruct((N, D_OUT), jnp.float32),
            jax.ShapeDtypeStruct((N, D_OUT), jnp.float32),
        ],
    )(q, h, dinv, Wmu, bmu, Wls, bls)


def kernel(x, edge_index, W1, b1, Wmu, bmu, Wls, bls):
    src = edge_index[0].reshape(NW, K, C)
    dst = edge_index[1].reshape(NW, K, C)

    deg_parts = _deg_call(dst)
    xw = _mm(x, W1)

    d0 = lax.slice(deg_parts, (0, 0, 0), (1, N, 1)).reshape(N, 1)
    d1 = lax.slice(deg_parts, (1, 0, 0), (2, N, 1)).reshape(N, 1)
    y1, dinv = _norm(d0, d1, xw)

    p = _agg_call(y1, src, dst)
    h, y2 = _layer1(p[:, :, :N], xw, dinv, b1.reshape(1, D_H))

    q = _agg_call(y2, src, dst)
    mu, ls = _heads(q[:, :, :N], h, dinv, Wmu, bmu.reshape(1, D_OUT),
                    Wls, bls.reshape(1, D_OUT))
    return (mu, ls)


# speedup vs baseline: 18.5333x; 1.1840x over previous
import jax
import jax.numpy as jnp
from jax import lax
from jax.experimental import pallas as pl
from jax.experimental.pallas import tpu as pltpu
from jax.experimental.pallas import tpu_sc as plsc

N = 10000
NPAD = 10240
E = 320000
D_IN = 128
D_H = 128
D_OUT = 64
DHF = D_H // 2

NC = 2
NS = 16
NW = NC * NS
EPW = E // NW
C = 80
K = EPW // C
RPT = NPAD // NS

_sc_mesh = plsc.VectorSubcoreMesh(core_axis_name="c", subcore_axis_name="s")


def _deg_body(dst_hbm, out_hbm, dst_v, ones_v, zbuf, acc):
    cid = lax.axis_index("c")
    sid = lax.axis_index("s")
    wid = sid * NC + cid

    one16 = jnp.ones((16,), jnp.float32)
    zero16 = jnp.zeros((16,), jnp.float32)

    @pl.loop(0, C)
    def _(i):
        ones_v[i, :] = one16

    @pl.loop(0, RPT)
    def _(i):
        zbuf[i, :] = zero16

    pltpu.sync_copy(dst_hbm.at[wid], dst_v)
    pltpu.sync_copy(zbuf, acc.at[pl.ds(sid * RPT, RPT)])
    plsc.subcore_barrier()

    @pl.loop(0, K)
    def _(j):
        pltpu.sync_copy(ones_v, acc.at[dst_v.at[j]], add=True)

    plsc.subcore_barrier()
    pltpu.sync_copy(acc.at[pl.ds(sid * RPT, RPT)],
                    out_hbm.at[cid, pl.ds(sid * RPT, RPT)])


_deg_call = pl.kernel(
    _deg_body,
    out_type=jax.ShapeDtypeStruct((NC, NPAD, 16), jnp.float32),
    mesh=_sc_mesh,
    scratch_types=[
        pltpu.VMEM((K, C), jnp.int32),
        pltpu.VMEM((C, 16), jnp.float32),
        pltpu.VMEM((RPT, 16), jnp.float32),
        pltpu.VMEM_SHARED((NPAD, 16), jnp.float32),
    ],
    compiler_params=pltpu.CompilerParams(use_tc_tiling_on_sc=False),
)


def _agg_body(y_hbm, src_hbm, dst_hbm, out_hbm, src_v, dst_v, rows0, rows1,
              zbuf, acc, sem0, sem1):
    cid = lax.axis_index("c")
    sid = lax.axis_index("s")
    wid = sid * NC + cid

    zero16 = jnp.zeros((16,), jnp.float32)

    @pl.loop(0, RPT)
    def _(i):
        @pl.loop(0, DHF, step=16)
        def _(c0):
            zbuf[i, pl.ds(c0, 16)] = zero16

    pltpu.sync_copy(src_hbm.at[wid], src_v)
    pltpu.sync_copy(dst_hbm.at[wid], dst_v)

    for half in range(2):
        pltpu.sync_copy(zbuf, acc.at[pl.ds(sid * RPT, RPT)])
        plsc.subcore_barrier()

        y_half = y_hbm.at[half]

        pltpu.async_copy(y_half.at[src_v.at[0]], rows0, sem0)

        @pl.loop(0, (K - 1) // 2)
        def _(i):
            j = 2 * i
            pltpu.make_async_copy(y_half.at[src_v.at[j]], rows0, sem0).wait()
            pltpu.async_copy(y_half.at[src_v.at[j + 1]], rows1, sem1)
            pltpu.sync_copy(rows0, acc.at[dst_v.at[j]], add=True)
            pltpu.make_async_copy(y_half.at[src_v.at[j + 1]], rows1, sem1).wait()
            pltpu.async_copy(y_half.at[src_v.at[j + 2]], rows0, sem0)
            pltpu.sync_copy(rows1, acc.at[dst_v.at[j + 1]], add=True)

        pltpu.make_async_copy(y_half.at[src_v.at[K - 1]], rows0, sem0).wait()
        pltpu.sync_copy(rows0, acc.at[dst_v.at[K - 1]], add=True)

        plsc.subcore_barrier()
        pltpu.sync_copy(acc.at[pl.ds(sid * RPT, RPT)],
                        out_hbm.at[cid, half, pl.ds(sid * RPT, RPT)])


_agg_call = pl.kernel(
    _agg_body,
    out_type=jax.ShapeDtypeStruct((NC, 2, NPAD, DHF), jnp.float32),
    mesh=_sc_mesh,
    scratch_types=[
        pltpu.VMEM((K, C), jnp.int32),
        pltpu.VMEM((K, C), jnp.int32),
        pltpu.VMEM((C, DHF), jnp.float32),
        pltpu.VMEM((C, DHF), jnp.float32),
        pltpu.VMEM((RPT, DHF), jnp.float32),
        pltpu.VMEM_SHARED((NPAD, DHF), jnp.float32),
        pltpu.SemaphoreType.DMA,
        pltpu.SemaphoreType.DMA,
    ],
    compiler_params=pltpu.CompilerParams(use_tc_tiling_on_sc=False),
)


BR = 1000


def _mm_body(x_ref, w_ref, o_ref):
    o_ref[...] = jnp.dot(x_ref[...], w_ref[...],
                         preferred_element_type=jnp.float32)


def _mm(x, w):
    n, d = x.shape
    return pl.pallas_call(
        _mm_body,
        grid=(n // BR,),
        in_specs=[
            pl.BlockSpec((BR, d), lambda i: (i, 0)),
            pl.BlockSpec((d, w.shape[1]), lambda i: (0, 0)),
        ],
        out_specs=pl.BlockSpec((BR, w.shape[1]), lambda i: (i, 0)),
        out_shape=jax.ShapeDtypeStruct((n, w.shape[1]), jnp.float32),
    )(x, w)


def _norm_body(d0_ref, d1_ref, xw_ref, y_ref, dinv_ref):
    deg = 1.0 + d0_ref[...] + d1_ref[...]
    dinv = lax.rsqrt(deg)
    dinv_ref[...] = dinv
    y = dinv * xw_ref[...]
    y_ref[0] = y[:, :DHF]
    y_ref[1] = y[:, DHF:]


def _norm(d0, d1, xw):
    return pl.pallas_call(
        _norm_body,
        grid=(N // BR,),
        in_specs=[
            pl.BlockSpec((BR, 1), lambda i: (i, 0)),
            pl.BlockSpec((BR, 1), lambda i: (i, 0)),
            pl.BlockSpec((BR, D_H), lambda i: (i, 0)),
        ],
        out_specs=[
            pl.BlockSpec((2, BR, DHF), lambda i: (0, i, 0)),
            pl.BlockSpec((BR, 1), lambda i: (i, 0)),
        ],
        out_shape=[
            jax.ShapeDtypeStruct((2, N, DHF), jnp.float32),
            jax.ShapeDtypeStruct((N, 1), jnp.float32),
        ],
    )(d0, d1, xw)


def _layer1_body(p_ref, xw_ref, dinv_ref, b_ref, h_ref, y2_ref):
    dinv = dinv_ref[...]
    agg = jnp.concatenate(
        [p_ref[0, 0] + p_ref[1, 0], p_ref[0, 1] + p_ref[1, 1]], axis=-1)
    pre = dinv * agg + (dinv * dinv) * xw_ref[...]
    h = jnp.maximum(pre + b_ref[...], 0.0)
    h_ref[...] = h
    y2 = dinv * h
    y2_ref[0] = y2[:, :DHF]
    y2_ref[1] = y2[:, DHF:]


def _layer1(p, xw, dinv, b1):
    return pl.pallas_call(
        _layer1_body,
        grid=(N // BR,),
        in_specs=[
            pl.BlockSpec((NC, 2, BR, DHF), lambda i: (0, 0, i, 0)),
            pl.BlockSpec((BR, D_H), lambda i: (i, 0)),
            pl.BlockSpec((BR, 1), lambda i: (i, 0)),
            pl.BlockSpec((1, D_H), lambda i: (0, 0)),
        ],
        out_specs=[
            pl.BlockSpec((BR, D_H), lambda i: (i, 0)),
            pl.BlockSpec((2, BR, DHF), lambda i: (0, i, 0)),
        ],
        out_shape=[
            jax.ShapeDtypeStruct((N, D_H), jnp.float32),
            jax.ShapeDtypeStruct((2, N, DHF), jnp.float32),
        ],
    )(p, xw, dinv, b1)


def _heads_body(q_ref, h_ref, dinv_ref, wmu_ref, bmu_ref, wls_ref, bls_ref,
                mu_ref, ls_ref):
    dinv = dinv_ref[...]
    agg = jnp.concatenate(
        [q_ref[0, 0] + q_ref[1, 0], q_ref[0, 1] + q_ref[1, 1]], axis=-1)
    ah = dinv * agg + (dinv * dinv) * h_ref[...]
    mu_ref[...] = jnp.dot(ah, wmu_ref[...],
                          preferred_element_type=jnp.float32) + bmu_ref[...]
    ls_ref[...] = jnp.dot(ah, wls_ref[...],
                          preferred_element_type=jnp.float32) + bls_ref[...]


def _heads(q, h, dinv, Wmu, bmu, Wls, bls):
    return pl.pallas_call(
        _heads_body,
        grid=(N // BR,),
        in_specs=[
            pl.BlockSpec((NC, 2, BR, DHF), lambda i: (0, 0, i, 0)),
            pl.BlockSpec((BR, D_H), lambda i: (i, 0)),
            pl.BlockSpec((BR, 1), lambda i: (i, 0)),
            pl.BlockSpec((D_H, D_OUT), lambda i: (0, 0)),
            pl.BlockSpec((1, D_OUT), lambda i: (0, 0)),
            pl.BlockSpec((D_H, D_OUT), lambda i: (0, 0)),
            pl.BlockSpec((1, D_OUT), lambda i: (0, 0)),
        ],
        out_specs=[
            pl.BlockSpec((BR, D_OUT), lambda i: (i, 0)),
            pl.BlockSpec((BR, D_OUT), lambda i: (i, 0)),
        ],
        out_shape=[
            jax.ShapeDtypeStruct((N, D_OUT), jnp.float32),
            jax.ShapeDtypeStruct((N, D_OUT), jnp.float32),
        ],
    )(q, h, dinv, Wmu, bmu, Wls, bls)


def kernel(x, edge_index, W1, b1, Wmu, bmu, Wls, bls):
    src = edge_index[0].reshape(NW, K, C)
    dst = edge_index[1].reshape(NW, K, C)

    deg_parts = _deg_call(dst)
    xw = _mm(x, W1)

    d0 = lax.slice(deg_parts, (0, 0, 0), (1, N, 1)).reshape(N, 1)
    d1 = lax.slice(deg_parts, (1, 0, 0), (2, N, 1)).reshape(N, 1)
    y1, dinv = _norm(d0, d1, xw)

    p = _agg_call(y1, src, dst)
    h, y2 = _layer1(p[:, :, :N], xw, dinv, b1.reshape(1, D_H))

    q = _agg_call(y2, src, dst)
    mu, ls = _heads(q[:, :, :N], h, dinv, Wmu, bmu.reshape(1, D_OUT),
                    Wls, bls.reshape(1, D_OUT))
    return (mu, ls)


# speedup vs baseline: 19.4121x; 1.0474x over previous
import jax
import jax.numpy as jnp
from jax import lax
from jax.experimental import pallas as pl
from jax.experimental.pallas import tpu as pltpu
from jax.experimental.pallas import tpu_sc as plsc

N = 10000
NPAD = 10240
E = 320000
D_IN = 128
D_H = 128
D_OUT = 64
DHF = D_H // 2

NC = 2
NS = 16
NW = NC * NS
EPW = E // NW
C = 80
K = EPW // C
RPT = NPAD // NS

_sc_mesh = plsc.VectorSubcoreMesh(core_axis_name="c", subcore_axis_name="s")


def _deg_body(dst_hbm, out_hbm, dst_v, ones_v, zbuf, acc):
    cid = lax.axis_index("c")
    sid = lax.axis_index("s")
    wid = sid * NC + cid

    one16 = jnp.ones((16,), jnp.float32)
    zero16 = jnp.zeros((16,), jnp.float32)

    @pl.loop(0, C)
    def _(i):
        ones_v[i, :] = one16

    @pl.loop(0, RPT)
    def _(i):
        zbuf[i, :] = zero16

    pltpu.sync_copy(dst_hbm.at[wid], dst_v)
    pltpu.sync_copy(zbuf, acc.at[pl.ds(sid * RPT, RPT)])
    plsc.subcore_barrier()

    @pl.loop(0, K)
    def _(j):
        pltpu.sync_copy(ones_v, acc.at[dst_v.at[j]], add=True)

    plsc.subcore_barrier()
    pltpu.sync_copy(acc.at[pl.ds(sid * RPT, RPT)],
                    out_hbm.at[cid, pl.ds(sid * RPT, RPT)])


_deg_call = pl.kernel(
    _deg_body,
    out_type=jax.ShapeDtypeStruct((NC, NPAD, 16), jnp.float32),
    mesh=_sc_mesh,
    scratch_types=[
        pltpu.VMEM((K, C), jnp.int32),
        pltpu.VMEM((C, 16), jnp.float32),
        pltpu.VMEM((RPT, 16), jnp.float32),
        pltpu.VMEM_SHARED((NPAD, 16), jnp.float32),
    ],
    compiler_params=pltpu.CompilerParams(use_tc_tiling_on_sc=False),
)


def _agg_body(y_hbm, src_hbm, dst_hbm, out_hbm, src_v, dst_v, rows0, rows1,
              zbuf, acc, sem0, sem1):
    cid = lax.axis_index("c")
    sid = lax.axis_index("s")
    wid = sid * NC + cid

    zero16 = jnp.zeros((16,), jnp.float32)

    @pl.loop(0, RPT)
    def _(i):
        @pl.loop(0, DHF, step=16)
        def _(c0):
            zbuf[i, pl.ds(c0, 16)] = zero16

    pltpu.sync_copy(src_hbm.at[wid], src_v)
    pltpu.sync_copy(dst_hbm.at[wid], dst_v)

    for half in range(2):
        pltpu.sync_copy(zbuf, acc.at[pl.ds(sid * RPT, RPT)])
        plsc.subcore_barrier()

        y_half = y_hbm.at[half]

        pltpu.async_copy(y_half.at[src_v.at[0]], rows0, sem0)

        @pl.loop(0, (K - 1) // 2)
        def _(i):
            j = 2 * i
            pltpu.make_async_copy(y_half.at[src_v.at[j]], rows0, sem0).wait()
            pltpu.async_copy(y_half.at[src_v.at[j + 1]], rows1, sem1)
            pltpu.sync_copy(rows0, acc.at[dst_v.at[j]], add=True)
            pltpu.make_async_copy(y_half.at[src_v.at[j + 1]], rows1, sem1).wait()
            pltpu.async_copy(y_half.at[src_v.at[j + 2]], rows0, sem0)
            pltpu.sync_copy(rows1, acc.at[dst_v.at[j + 1]], add=True)

        pltpu.make_async_copy(y_half.at[src_v.at[K - 1]], rows0, sem0).wait()
        pltpu.sync_copy(rows0, acc.at[dst_v.at[K - 1]], add=True)

        plsc.subcore_barrier()
        pltpu.sync_copy(acc.at[pl.ds(sid * RPT, RPT)],
                        out_hbm.at[cid, half, pl.ds(sid * RPT, RPT)])


_agg_call = pl.kernel(
    _agg_body,
    out_type=jax.ShapeDtypeStruct((NC, 2, NPAD, DHF), jnp.float32),
    mesh=_sc_mesh,
    scratch_types=[
        pltpu.VMEM((K, C), jnp.int32),
        pltpu.VMEM((K, C), jnp.int32),
        pltpu.VMEM((C, DHF), jnp.float32),
        pltpu.VMEM((C, DHF), jnp.float32),
        pltpu.VMEM((RPT, DHF), jnp.float32),
        pltpu.VMEM_SHARED((NPAD, DHF), jnp.float32),
        pltpu.SemaphoreType.DMA,
        pltpu.SemaphoreType.DMA,
    ],
    compiler_params=pltpu.CompilerParams(use_tc_tiling_on_sc=False),
)


BR = 1000


def _mm_body(x_ref, w_ref, o_ref):
    o_ref[...] = jnp.dot(x_ref[...], w_ref[...],
                         preferred_element_type=jnp.float32)


def _mm(x, w):
    n, d = x.shape
    return pl.pallas_call(
        _mm_body,
        grid=(n // BR,),
        in_specs=[
            pl.BlockSpec((BR, d), lambda i: (i, 0)),
            pl.BlockSpec((d, w.shape[1]), lambda i: (0, 0)),
        ],
        out_specs=pl.BlockSpec((BR, w.shape[1]), lambda i: (i, 0)),
        out_shape=jax.ShapeDtypeStruct((n, w.shape[1]), jnp.float32),
    )(x, w)


def _norm_body(d0_ref, d1_ref, xw_ref, y_ref, dinv_ref):
    deg = 1.0 + d0_ref[...] + d1_ref[...]
    dinv = lax.rsqrt(deg)
    dinv_ref[...] = dinv
    y = dinv * xw_ref[...]
    y_ref[0] = y[:, :DHF]
    y_ref[1] = y[:, DHF:]


def _norm(d0, d1, xw):
    return pl.pallas_call(
        _norm_body,
        grid=(N // BR,),
        in_specs=[
            pl.BlockSpec((BR, 1), lambda i: (i, 0)),
            pl.BlockSpec((BR, 1), lambda i: (i, 0)),
            pl.BlockSpec((BR, D_H), lambda i: (i, 0)),
        ],
        out_specs=[
            pl.BlockSpec((2, BR, DHF), lambda i: (0, i, 0)),
            pl.BlockSpec((BR, 1), lambda i: (i, 0)),
        ],
        out_shape=[
            jax.ShapeDtypeStruct((2, N, DHF), jnp.float32),
            jax.ShapeDtypeStruct((N, 1), jnp.float32),
        ],
    )(d0, d1, xw)


def _layer1_body(p_ref, xw_ref, dinv_ref, b_ref, h_ref, y2_ref):
    dinv = dinv_ref[...]
    agg = jnp.concatenate(
        [p_ref[0, 0] + p_ref[1, 0], p_ref[0, 1] + p_ref[1, 1]], axis=-1)
    pre = dinv * agg + (dinv * dinv) * xw_ref[...]
    h = jnp.maximum(pre + b_ref[...], 0.0)
    h_ref[...] = h
    y2 = dinv * h
    y2_ref[0] = y2[:, :DHF]
    y2_ref[1] = y2[:, DHF:]


def _layer1(p, xw, dinv, b1):
    return pl.pallas_call(
        _layer1_body,
        grid=(N // BR,),
        in_specs=[
            pl.BlockSpec((NC, 2, BR, DHF), lambda i: (0, 0, i, 0)),
            pl.BlockSpec((BR, D_H), lambda i: (i, 0)),
            pl.BlockSpec((BR, 1), lambda i: (i, 0)),
            pl.BlockSpec((1, D_H), lambda i: (0, 0)),
        ],
        out_specs=[
            pl.BlockSpec((BR, D_H), lambda i: (i, 0)),
            pl.BlockSpec((2, BR, DHF), lambda i: (0, i, 0)),
        ],
        out_shape=[
            jax.ShapeDtypeStruct((N, D_H), jnp.float32),
            jax.ShapeDtypeStruct((2, N, DHF), jnp.float32),
        ],
    )(p, xw, dinv, b1)


def _heads_body(q_ref, h_ref, dinv_ref, wmu_ref, bmu_ref, wls_ref, bls_ref,
                mu_ref, ls_ref):
    dinv = dinv_ref[...]
    agg = jnp.concatenate(
        [q_ref[0, 0] + q_ref[1, 0], q_ref[0, 1] + q_ref[1, 1]], axis=-1)
    ah = dinv * agg + (dinv * dinv) * h_ref[...]
    mu_ref[...] = jnp.dot(ah, wmu_ref[...],
                          preferred_element_type=jnp.float32) + bmu_ref[...]
    ls_ref[...] = jnp.dot(ah, wls_ref[...],
                          preferred_element_type=jnp.float32) + bls_ref[...]


def _heads(q, h, dinv, Wmu, bmu, Wls, bls):
    return pl.pallas_call(
        _heads_body,
        grid=(N // BR,),
        in_specs=[
            pl.BlockSpec((NC, 2, BR, DHF), lambda i: (0, 0, i, 0)),
            pl.BlockSpec((BR, D_H), lambda i: (i, 0)),
            pl.BlockSpec((BR, 1), lambda i: (i, 0)),
            pl.BlockSpec((D_H, D_OUT), lambda i: (0, 0)),
            pl.BlockSpec((1, D_OUT), lambda i: (0, 0)),
            pl.BlockSpec((D_H, D_OUT), lambda i: (0, 0)),
            pl.BlockSpec((1, D_OUT), lambda i: (0, 0)),
        ],
        out_specs=[
            pl.BlockSpec((BR, D_OUT), lambda i: (i, 0)),
            pl.BlockSpec((BR, D_OUT), lambda i: (i, 0)),
        ],
        out_shape=[
            jax.ShapeDtypeStruct((N, D_OUT), jnp.float32),
            jax.ShapeDtypeStruct((N, D_OUT), jnp.float32),
        ],
    )(q, h, dinv, Wmu, bmu, Wls, bls)


def kernel(x, edge_index, W1, b1, Wmu, bmu, Wls, bls):
    src = edge_index[0].reshape(NW, K, C)
    dst = edge_index[1].reshape(NW, K, C)

    deg_parts = _deg_call(dst)
    xw = _mm(x, W1)

    d0 = lax.slice(deg_parts, (0, 0, 0), (1, N, 1)).reshape(N, 1)
    d1 = lax.slice(deg_parts, (1, 0, 0), (2, N, 1)).reshape(N, 1)
    y1, dinv = _norm(d0, d1, xw)

    p = _agg_call(y1, src, dst)
    h, y2 = _layer1(p, xw, dinv, b1.reshape(1, D_H))

    q = _agg_call(y2, src, dst)
    mu, ls = _heads(q, h, dinv, Wmu, bmu.reshape(1, D_OUT),
                    Wls, bls.reshape(1, D_OUT))
    return (mu, ls)


# speedup vs baseline: 31.6559x; 1.6307x over previous
import jax
import jax.numpy as jnp
from jax import lax
from jax.experimental import pallas as pl
from jax.experimental.pallas import tpu as pltpu
from jax.experimental.pallas import tpu_sc as plsc

N = 10000
NPAD = 10240
E = 320000
D_IN = 128
D_H = 128
D_OUT = 64
DHF = D_H // 2

NC = 2
NS = 16
NW = NC * NS
EPW = E // NW
C = 80
K = EPW // C
RPT = NPAD // NS

_sc_mesh = plsc.VectorSubcoreMesh(core_axis_name="c", subcore_axis_name="s")


def _deg_body(dst_hbm, out_hbm, dst_v, ones_v, zbuf, acc):
    cid = lax.axis_index("c")
    sid = lax.axis_index("s")
    wid = sid * NC + cid

    one16 = jnp.ones((16,), jnp.float32)
    zero16 = jnp.zeros((16,), jnp.float32)

    @pl.loop(0, C)
    def _(i):
        ones_v[i, :] = one16

    @pl.loop(0, RPT)
    def _(i):
        zbuf[i, :] = zero16

    pltpu.sync_copy(dst_hbm.at[wid], dst_v)
    pltpu.sync_copy(zbuf, acc.at[pl.ds(sid * RPT, RPT)])
    plsc.subcore_barrier()

    @pl.loop(0, K)
    def _(j):
        pltpu.sync_copy(ones_v, acc.at[dst_v.at[j]], add=True)

    plsc.subcore_barrier()
    pltpu.sync_copy(acc.at[pl.ds(sid * RPT, RPT)],
                    out_hbm.at[cid, pl.ds(sid * RPT, RPT)])


_deg_call = pl.kernel(
    _deg_body,
    out_type=jax.ShapeDtypeStruct((NC, NPAD, 16), jnp.float32),
    mesh=_sc_mesh,
    scratch_types=[
        pltpu.VMEM((K, C), jnp.int32),
        pltpu.VMEM((C, 16), jnp.float32),
        pltpu.VMEM((RPT, 16), jnp.float32),
        pltpu.VMEM_SHARED((NPAD, 16), jnp.float32),
    ],
    compiler_params=pltpu.CompilerParams(use_tc_tiling_on_sc=False),
)


NBUF = 4


def _agg_body(y_hbm, src_hbm, dst_hbm, out_hbm, src_v, dst_v,
              rows0, rows1, rows2, rows3, zbuf, acc,
              sem0, sem1, sem2, sem3):
    cid = lax.axis_index("c")
    sid = lax.axis_index("s")
    wid = sid * NC + cid
    rows = (rows0, rows1, rows2, rows3)
    sems = (sem0, sem1, sem2, sem3)

    zero16 = jnp.zeros((16,), jnp.float32)

    @pl.loop(0, RPT)
    def _(i):
        @pl.loop(0, DHF, step=16)
        def _(c0):
            zbuf[i, pl.ds(c0, 16)] = zero16

    pltpu.sync_copy(src_hbm.at[wid], src_v)
    pltpu.sync_copy(dst_hbm.at[wid], dst_v)

    for half in range(2):
        pltpu.sync_copy(zbuf, acc.at[pl.ds(sid * RPT, RPT)])
        plsc.subcore_barrier()

        y_half = y_hbm.at[half]

        for b in range(NBUF):
            pltpu.async_copy(y_half.at[src_v.at[b]], rows[b], sems[b])

        @pl.loop(0, (K - NBUF) // NBUF)
        def _(i):
            j = NBUF * i
            for b in range(NBUF):
                pltpu.make_async_copy(
                    y_half.at[src_v.at[j + b]], rows[b], sems[b]).wait()
                pltpu.sync_copy(rows[b], acc.at[dst_v.at[j + b]], add=True)
                pltpu.async_copy(
                    y_half.at[src_v.at[j + b + NBUF]], rows[b], sems[b])

        base = ((K - NBUF) // NBUF) * NBUF
        for b in range(NBUF):
            pltpu.make_async_copy(
                y_half.at[src_v.at[base + b]], rows[b], sems[b]).wait()
            pltpu.sync_copy(rows[b], acc.at[dst_v.at[base + b]], add=True)
        pltpu.sync_copy(y_half.at[src_v.at[K - 1]], rows0)
        pltpu.sync_copy(rows0, acc.at[dst_v.at[K - 1]], add=True)

        plsc.subcore_barrier()
        pltpu.sync_copy(acc.at[pl.ds(sid * RPT, RPT)],
                        out_hbm.at[cid, half, pl.ds(sid * RPT, RPT)])


_agg_call = pl.kernel(
    _agg_body,
    out_type=jax.ShapeDtypeStruct((NC, 2, NPAD, DHF), jnp.float32),
    mesh=_sc_mesh,
    scratch_types=[
        pltpu.VMEM((K, C), jnp.int32),
        pltpu.VMEM((K, C), jnp.int32),
        pltpu.VMEM((C, DHF), jnp.float32),
        pltpu.VMEM((C, DHF), jnp.float32),
        pltpu.VMEM((C, DHF), jnp.float32),
        pltpu.VMEM((C, DHF), jnp.float32),
        pltpu.VMEM((RPT, DHF), jnp.float32),
        pltpu.VMEM_SHARED((NPAD, DHF), jnp.float32),
        pltpu.SemaphoreType.DMA,
        pltpu.SemaphoreType.DMA,
        pltpu.SemaphoreType.DMA,
        pltpu.SemaphoreType.DMA,
    ],
    compiler_params=pltpu.CompilerParams(use_tc_tiling_on_sc=False),
)


BR = 1000


def _mm_body(x_ref, w_ref, o_ref):
    o_ref[...] = jnp.dot(x_ref[...], w_ref[...],
                         preferred_element_type=jnp.float32)


def _mm(x, w):
    n, d = x.shape
    return pl.pallas_call(
        _mm_body,
        grid=(n // BR,),
        in_specs=[
            pl.BlockSpec((BR, d), lambda i: (i, 0)),
            pl.BlockSpec((d, w.shape[1]), lambda i: (0, 0)),
        ],
        out_specs=pl.BlockSpec((BR, w.shape[1]), lambda i: (i, 0)),
        out_shape=jax.ShapeDtypeStruct((n, w.shape[1]), jnp.float32),
    )(x, w)


def _norm_body(d0_ref, d1_ref, xw_ref, y_ref, dinv_ref):
    deg = 1.0 + d0_ref[...] + d1_ref[...]
    dinv = lax.rsqrt(deg)
    dinv_ref[...] = dinv
    y = dinv * xw_ref[...]
    y_ref[0] = y[:, :DHF]
    y_ref[1] = y[:, DHF:]


def _norm(d0, d1, xw):
    return pl.pallas_call(
        _norm_body,
        grid=(N // BR,),
        in_specs=[
            pl.BlockSpec((BR, 1), lambda i: (i, 0)),
            pl.BlockSpec((BR, 1), lambda i: (i, 0)),
            pl.BlockSpec((BR, D_H), lambda i: (i, 0)),
        ],
        out_specs=[
            pl.BlockSpec((2, BR, DHF), lambda i: (0, i, 0)),
            pl.BlockSpec((BR, 1), lambda i: (i, 0)),
        ],
        out_shape=[
            jax.ShapeDtypeStruct((2, N, DHF), jnp.float32),
            jax.ShapeDtypeStruct((N, 1), jnp.float32),
        ],
    )(d0, d1, xw)


def _layer1_body(p_ref, xw_ref, dinv_ref, b_ref, h_ref, y2_ref):
    dinv = dinv_ref[...]
    agg = jnp.concatenate(
        [p_ref[0, 0] + p_ref[1, 0], p_ref[0, 1] + p_ref[1, 1]], axis=-1)
    pre = dinv * agg + (dinv * dinv) * xw_ref[...]
    h = jnp.maximum(pre + b_ref[...], 0.0)
    h_ref[...] = h
    y2 = dinv * h
    y2_ref[0] = y2[:, :DHF]
    y2_ref[1] = y2[:, DHF:]


def _layer1(p, xw, dinv, b1):
    return pl.pallas_call(
        _layer1_body,
        grid=(N // BR,),
        in_specs=[
            pl.BlockSpec((NC, 2, BR, DHF), lambda i: (0, 0, i, 0)),
            pl.BlockSpec((BR, D_H), lambda i: (i, 0)),
            pl.BlockSpec((BR, 1), lambda i: (i, 0)),
            pl.BlockSpec((1, D_H), lambda i: (0, 0)),
        ],
        out_specs=[
            pl.BlockSpec((BR, D_H), lambda i: (i, 0)),
            pl.BlockSpec((2, BR, DHF), lambda i: (0, i, 0)),
        ],
        out_shape=[
            jax.ShapeDtypeStruct((N, D_H), jnp.float32),
            jax.ShapeDtypeStruct((2, N, DHF), jnp.float32),
        ],
    )(p, xw, dinv, b1)


def _heads_body(q_ref, h_ref, dinv_ref, wmu_ref, bmu_ref, wls_ref, bls_ref,
                mu_ref, ls_ref):
    dinv = dinv_ref[...]
    agg = jnp.concatenate(
        [q_ref[0, 0] + q_ref[1, 0], q_ref[0, 1] + q_ref[1, 1]], axis=-1)
    ah = dinv * agg + (dinv * dinv) * h_ref[...]
    mu_ref[...] = jnp.dot(ah, wmu_ref[...],
                          preferred_element_type=jnp.float32) + bmu_ref[...]
    ls_ref[...] = jnp.dot(ah, wls_ref[...],
                          preferred_element_type=jnp.float32) + bls_ref[...]


def _heads(q, h, dinv, Wmu, bmu, Wls, bls):
    return pl.pallas_call(
        _heads_body,
        grid=(N // BR,),
        in_specs=[
            pl.BlockSpec((NC, 2, BR, DHF), lambda i: (0, 0, i, 0)),
            pl.BlockSpec((BR, D_H), lambda i: (i, 0)),
            pl.BlockSpec((BR, 1), lambda i: (i, 0)),
            pl.BlockSpec((D_H, D_OUT), lambda i: (0, 0)),
            pl.BlockSpec((1, D_OUT), lambda i: (0, 0)),
            pl.BlockSpec((D_H, D_OUT), lambda i: (0, 0)),
            pl.BlockSpec((1, D_OUT), lambda i: (0, 0)),
        ],
        out_specs=[
            pl.BlockSpec((BR, D_OUT), lambda i: (i, 0)),
            pl.BlockSpec((BR, D_OUT), lambda i: (i, 0)),
        ],
        out_shape=[
            jax.ShapeDtypeStruct((N, D_OUT), jnp.float32),
            jax.ShapeDtypeStruct((N, D_OUT), jnp.float32),
        ],
    )(q, h, dinv, Wmu, bmu, Wls, bls)


def kernel(x, edge_index, W1, b1, Wmu, bmu, Wls, bls):
    src = edge_index[0].reshape(NW, K, C)
    dst = edge_index[1].reshape(NW, K, C)

    deg_parts = _deg_call(dst)
    xw = _mm(x, W1)

    d0 = lax.slice(deg_parts, (0, 0, 0), (1, N, 1)).reshape(N, 1)
    d1 = lax.slice(deg_parts, (1, 0, 0), (2, N, 1)).reshape(N, 1)
    y1, dinv = _norm(d0, d1, xw)

    p = _agg_call(y1, src, dst)
    h, y2 = _layer1(p, xw, dinv, b1.reshape(1, D_H))

    q = _agg_call(y2, src, dst)
    mu, ls = _heads(q, h, dinv, Wmu, bmu.reshape(1, D_OUT),
                    Wls, bls.reshape(1, D_OUT))
    return (mu, ls)


# speedup vs baseline: 35.1932x; 1.1117x over previous
import jax
import jax.numpy as jnp
from jax import lax
from jax.experimental import pallas as pl
from jax.experimental.pallas import tpu as pltpu
from jax.experimental.pallas import tpu_sc as plsc

N = 10000
NPAD = 10240
E = 320000
D_IN = 128
D_H = 128
D_OUT = 64
DHF = D_H // 2

NC = 2
NS = 16
NW = NC * NS
EPW = E // NW
C = 80
K = EPW // C
RPT = NPAD // NS

_sc_mesh = plsc.VectorSubcoreMesh(core_axis_name="c", subcore_axis_name="s")


def _deg_body(dst_hbm, out_hbm, dst_v, ones_v, zbuf, acc):
    cid = lax.axis_index("c")
    sid = lax.axis_index("s")
    wid = sid * NC + cid

    one16 = jnp.ones((16,), jnp.float32)
    zero16 = jnp.zeros((16,), jnp.float32)

    @pl.loop(0, C)
    def _(i):
        ones_v[i, :] = one16

    @pl.loop(0, RPT)
    def _(i):
        zbuf[i, :] = zero16

    pltpu.sync_copy(dst_hbm.at[wid], dst_v)
    pltpu.sync_copy(zbuf, acc.at[pl.ds(sid * RPT, RPT)])
    plsc.subcore_barrier()

    @pl.loop(0, K)
    def _(j):
        pltpu.sync_copy(ones_v, acc.at[dst_v.at[j]], add=True)

    plsc.subcore_barrier()
    pltpu.sync_copy(acc.at[pl.ds(sid * RPT, RPT)],
                    out_hbm.at[cid, pl.ds(sid * RPT, RPT)])


_deg_call = pl.kernel(
    _deg_body,
    out_type=jax.ShapeDtypeStruct((NC, NPAD, 16), jnp.float32),
    mesh=_sc_mesh,
    scratch_types=[
        pltpu.VMEM((K, C), jnp.int32),
        pltpu.VMEM((C, 16), jnp.float32),
        pltpu.VMEM((RPT, 16), jnp.float32),
        pltpu.VMEM_SHARED((NPAD, 16), jnp.float32),
    ],
    compiler_params=pltpu.CompilerParams(use_tc_tiling_on_sc=False),
)


NBUF = 4


def _agg_body(y_hbm, src_hbm, dst_hbm, out_hbm, src_v, dst_v,
              rows0, rows1, rows2, rows3, zbuf, acc,
              sem0, sem1, sem2, sem3):
    cid = lax.axis_index("c")
    sid = lax.axis_index("s")
    wid = sid * NC + cid
    rows = (rows0, rows1, rows2, rows3)
    sems = (sem0, sem1, sem2, sem3)

    zero16 = jnp.zeros((16,), jnp.float32)

    @pl.loop(0, RPT)
    def _(i):
        @pl.loop(0, DHF, step=16)
        def _(c0):
            zbuf[i, pl.ds(c0, 16)] = zero16

    pltpu.sync_copy(src_hbm.at[wid], src_v)
    pltpu.sync_copy(dst_hbm.at[wid], dst_v)

    for half in range(2):
        pltpu.sync_copy(zbuf, acc.at[pl.ds(sid * RPT, RPT)])
        plsc.subcore_barrier()

        y_half = y_hbm.at[half]

        for b in range(NBUF):
            pltpu.async_copy(y_half.at[src_v.at[b]], rows[b], sems[b])

        @pl.loop(0, (K - NBUF) // NBUF)
        def _(i):
            j = NBUF * i
            for b in range(NBUF):
                pltpu.make_async_copy(
                    y_half.at[src_v.at[j + b]], rows[b], sems[b]).wait()
                pltpu.sync_copy(rows[b], acc.at[dst_v.at[j + b]], add=True)
                pltpu.async_copy(
                    y_half.at[src_v.at[j + b + NBUF]], rows[b], sems[b])

        base = ((K - NBUF) // NBUF) * NBUF
        for b in range(NBUF):
            pltpu.make_async_copy(
                y_half.at[src_v.at[base + b]], rows[b], sems[b]).wait()
            pltpu.sync_copy(rows[b], acc.at[dst_v.at[base + b]], add=True)
        pltpu.sync_copy(y_half.at[src_v.at[K - 1]], rows0)
        pltpu.sync_copy(rows0, acc.at[dst_v.at[K - 1]], add=True)

        plsc.subcore_barrier()
        pltpu.sync_copy(acc.at[pl.ds(sid * RPT, RPT)],
                        out_hbm.at[cid, pl.ds(sid * RPT, RPT),
                                   pl.ds(half * DHF, DHF)])


_agg_call = pl.kernel(
    _agg_body,
    out_type=jax.ShapeDtypeStruct((NC, NPAD, D_H), jnp.float32),
    mesh=_sc_mesh,
    scratch_types=[
        pltpu.VMEM((K, C), jnp.int32),
        pltpu.VMEM((K, C), jnp.int32),
        pltpu.VMEM((C, DHF), jnp.float32),
        pltpu.VMEM((C, DHF), jnp.float32),
        pltpu.VMEM((C, DHF), jnp.float32),
        pltpu.VMEM((C, DHF), jnp.float32),
        pltpu.VMEM((RPT, DHF), jnp.float32),
        pltpu.VMEM_SHARED((NPAD, DHF), jnp.float32),
        pltpu.SemaphoreType.DMA,
        pltpu.SemaphoreType.DMA,
        pltpu.SemaphoreType.DMA,
        pltpu.SemaphoreType.DMA,
    ],
    compiler_params=pltpu.CompilerParams(use_tc_tiling_on_sc=False),
)


BR = 1000


def _mm_body(x_ref, w_ref, o_ref):
    o_ref[...] = jnp.dot(x_ref[...], w_ref[...],
                         preferred_element_type=jnp.float32)


def _mm(x, w):
    n, d = x.shape
    return pl.pallas_call(
        _mm_body,
        grid=(n // BR,),
        in_specs=[
            pl.BlockSpec((BR, d), lambda i: (i, 0)),
            pl.BlockSpec((d, w.shape[1]), lambda i: (0, 0)),
        ],
        out_specs=pl.BlockSpec((BR, w.shape[1]), lambda i: (i, 0)),
        out_shape=jax.ShapeDtypeStruct((n, w.shape[1]), jnp.float32),
    )(x, w)


def _norm_body(d0_ref, d1_ref, xw_ref, y_ref, dinv_ref):
    deg = 1.0 + d0_ref[...] + d1_ref[...]
    dinv = lax.rsqrt(deg)
    dinv_ref[...] = dinv
    y = dinv * xw_ref[...]
    y_ref[0] = y[:, :DHF]
    y_ref[1] = y[:, DHF:]


def _norm(d0, d1, xw):
    return pl.pallas_call(
        _norm_body,
        grid=(N // BR,),
        in_specs=[
            pl.BlockSpec((BR, 1), lambda i: (i, 0)),
            pl.BlockSpec((BR, 1), lambda i: (i, 0)),
            pl.BlockSpec((BR, D_H), lambda i: (i, 0)),
        ],
        out_specs=[
            pl.BlockSpec((2, BR, DHF), lambda i: (0, i, 0)),
            pl.BlockSpec((BR, 1), lambda i: (i, 0)),
        ],
        out_shape=[
            jax.ShapeDtypeStruct((2, N, DHF), jnp.float32),
            jax.ShapeDtypeStruct((N, 1), jnp.float32),
        ],
    )(d0, d1, xw)


def _layer1_body(p_ref, xw_ref, dinv_ref, b_ref, h_ref, y2_ref):
    dinv = dinv_ref[...]
    agg = p_ref[0] + p_ref[1]
    pre = dinv * agg + (dinv * dinv) * xw_ref[...]
    h = jnp.maximum(pre + b_ref[...], 0.0)
    h_ref[...] = h
    y2 = dinv * h
    y2_ref[0] = y2[:, :DHF]
    y2_ref[1] = y2[:, DHF:]


def _layer1(p, xw, dinv, b1):
    return pl.pallas_call(
        _layer1_body,
        grid=(N // BR,),
        in_specs=[
            pl.BlockSpec((NC, BR, D_H), lambda i: (0, i, 0)),
            pl.BlockSpec((BR, D_H), lambda i: (i, 0)),
            pl.BlockSpec((BR, 1), lambda i: (i, 0)),
            pl.BlockSpec((1, D_H), lambda i: (0, 0)),
        ],
        out_specs=[
            pl.BlockSpec((BR, D_H), lambda i: (i, 0)),
            pl.BlockSpec((2, BR, DHF), lambda i: (0, i, 0)),
        ],
        out_shape=[
            jax.ShapeDtypeStruct((N, D_H), jnp.float32),
            jax.ShapeDtypeStruct((2, N, DHF), jnp.float32),
        ],
    )(p, xw, dinv, b1)


def _heads_body(q_ref, h_ref, dinv_ref, wmu_ref, bmu_ref, wls_ref, bls_ref,
                mu_ref, ls_ref):
    dinv = dinv_ref[...]
    ah = dinv * (q_ref[0] + q_ref[1]) + (dinv * dinv) * h_ref[...]
    mu_ref[...] = jnp.dot(ah, wmu_ref[...],
                          preferred_element_type=jnp.float32) + bmu_ref[...]
    ls_ref[...] = jnp.dot(ah, wls_ref[...],
                          preferred_element_type=jnp.float32) + bls_ref[...]


def _heads(q, h, dinv, Wmu, bmu, Wls, bls):
    return pl.pallas_call(
        _heads_body,
        grid=(N // BR,),
        in_specs=[
            pl.BlockSpec((NC, BR, D_H), lambda i: (0, i, 0)),
            pl.BlockSpec((BR, D_H), lambda i: (i, 0)),
            pl.BlockSpec((BR, 1), lambda i: (i, 0)),
            pl.BlockSpec((D_H, D_OUT), lambda i: (0, 0)),
            pl.BlockSpec((1, D_OUT), lambda i: (0, 0)),
            pl.BlockSpec((D_H, D_OUT), lambda i: (0, 0)),
            pl.BlockSpec((1, D_OUT), lambda i: (0, 0)),
        ],
        out_specs=[
            pl.BlockSpec((BR, D_OUT), lambda i: (i, 0)),
            pl.BlockSpec((BR, D_OUT), lambda i: (i, 0)),
        ],
        out_shape=[
            jax.ShapeDtypeStruct((N, D_OUT), jnp.float32),
            jax.ShapeDtypeStruct((N, D_OUT), jnp.float32),
        ],
    )(q, h, dinv, Wmu, bmu, Wls, bls)


def kernel(x, edge_index, W1, b1, Wmu, bmu, Wls, bls):
    src = edge_index[0].reshape(NW, K, C)
    dst = edge_index[1].reshape(NW, K, C)

    deg_parts = _deg_call(dst)
    xw = _mm(x, W1)

    d0 = lax.slice(deg_parts, (0, 0, 0), (1, N, 1)).reshape(N, 1)
    d1 = lax.slice(deg_parts, (1, 0, 0), (2, N, 1)).reshape(N, 1)
    y1, dinv = _norm(d0, d1, xw)

    p = _agg_call(y1, src, dst)
    h, y2 = _layer1(p, xw, dinv, b1.reshape(1, D_H))

    q = _agg_call(y2, src, dst)
    mu, ls = _heads(q, h, dinv, Wmu, bmu.reshape(1, D_OUT),
                    Wls, bls.reshape(1, D_OUT))
    return (mu, ls)


# speedup vs baseline: 37.9047x; 1.0770x over previous
import jax
import jax.numpy as jnp
from jax import lax
from jax.experimental import pallas as pl
from jax.experimental.pallas import tpu as pltpu
from jax.experimental.pallas import tpu_sc as plsc

N = 10000
NPAD = 10240
E = 320000
D_IN = 128
D_H = 128
D_OUT = 64
DHF = D_H // 2

NC = 2
NS = 16
NW = NC * NS
EPW = E // NW
C = 80
K = EPW // C
RPT = NPAD // NS

_sc_mesh = plsc.VectorSubcoreMesh(core_axis_name="c", subcore_axis_name="s")


def _deg_body(dst_hbm, out_hbm, dst_v, ones_v, zbuf, acc):
    cid = lax.axis_index("c")
    sid = lax.axis_index("s")
    wid = sid * NC + cid

    one16 = jnp.ones((16,), jnp.float32)
    zero16 = jnp.zeros((16,), jnp.float32)

    @pl.loop(0, C)
    def _(i):
        ones_v[i, :] = one16

    @pl.loop(0, RPT)
    def _(i):
        zbuf[i, :] = zero16

    pltpu.sync_copy(dst_hbm.at[wid], dst_v)
    pltpu.sync_copy(zbuf, acc.at[pl.ds(sid * RPT, RPT)])
    plsc.subcore_barrier()

    @pl.loop(0, K)
    def _(j):
        pltpu.sync_copy(ones_v, acc.at[dst_v.at[j]], add=True)

    plsc.subcore_barrier()
    pltpu.sync_copy(acc.at[pl.ds(sid * RPT, RPT)],
                    out_hbm.at[cid, pl.ds(sid * RPT, RPT)])


_deg_call = pl.kernel(
    _deg_body,
    out_type=jax.ShapeDtypeStruct((NC, NPAD, 16), jnp.float32),
    mesh=_sc_mesh,
    scratch_types=[
        pltpu.VMEM((K, C), jnp.int32),
        pltpu.VMEM((C, 16), jnp.float32),
        pltpu.VMEM((RPT, 16), jnp.float32),
        pltpu.VMEM_SHARED((NPAD, 16), jnp.float32),
    ],
    compiler_params=pltpu.CompilerParams(use_tc_tiling_on_sc=False),
)


NBUF = 4


def _agg_body(y_hbm, src_hbm, dst_hbm, out_hbm, src_v, dst_v,
              rows0, rows1, rows2, rows3, zbuf, acc,
              sem0, sem1, sem2, sem3):
    cid = lax.axis_index("c")
    sid = lax.axis_index("s")
    wid = sid * NC + cid
    rows = (rows0, rows1, rows2, rows3)
    sems = (sem0, sem1, sem2, sem3)

    zero16 = jnp.zeros((16,), jnp.float32)

    @pl.loop(0, RPT)
    def _(i):
        @pl.loop(0, DHF, step=16)
        def _(c0):
            zbuf[i, pl.ds(c0, 16)] = zero16

    pltpu.sync_copy(src_hbm.at[wid], src_v)
    pltpu.sync_copy(dst_hbm.at[wid], dst_v)

    one16i = jnp.ones((16,), jnp.int32)

    for half in range(2):
        if half == 1:
            @pl.loop(0, K)
            def _(k):
                @pl.loop(0, C, step=16)
                def _(c0):
                    src_v[k, pl.ds(c0, 16)] = src_v[k, pl.ds(c0, 16)] + one16i

        pltpu.sync_copy(zbuf, acc.at[pl.ds(sid * RPT, RPT)])
        plsc.subcore_barrier()

        y_half = y_hbm

        for b in range(NBUF):
            pltpu.async_copy(y_half.at[src_v.at[b]], rows[b], sems[b])

        @pl.loop(0, (K - NBUF) // NBUF)
        def _(i):
            j = NBUF * i
            for b in range(NBUF):
                pltpu.make_async_copy(
                    y_half.at[src_v.at[j + b]], rows[b], sems[b]).wait()
                pltpu.sync_copy(rows[b], acc.at[dst_v.at[j + b]], add=True)
                pltpu.async_copy(
                    y_half.at[src_v.at[j + b + NBUF]], rows[b], sems[b])

        base = ((K - NBUF) // NBUF) * NBUF
        for b in range(NBUF):
            pltpu.make_async_copy(
                y_half.at[src_v.at[base + b]], rows[b], sems[b]).wait()
            pltpu.sync_copy(rows[b], acc.at[dst_v.at[base + b]], add=True)
        pltpu.sync_copy(y_half.at[src_v.at[K - 1]], rows0)
        pltpu.sync_copy(rows0, acc.at[dst_v.at[K - 1]], add=True)

        plsc.subcore_barrier()
        pltpu.sync_copy(acc.at[pl.ds(sid * RPT, RPT)],
                        out_hbm.at[cid, pl.ds(sid * RPT, RPT),
                                   pl.ds(half * DHF, DHF)])


_agg_call = pl.kernel(
    _agg_body,
    out_type=jax.ShapeDtypeStruct((NC, NPAD, D_H), jnp.float32),
    mesh=_sc_mesh,
    scratch_types=[
        pltpu.VMEM((K, C), jnp.int32),
        pltpu.VMEM((K, C), jnp.int32),
        pltpu.VMEM((C, DHF), jnp.float32),
        pltpu.VMEM((C, DHF), jnp.float32),
        pltpu.VMEM((C, DHF), jnp.float32),
        pltpu.VMEM((C, DHF), jnp.float32),
        pltpu.VMEM((RPT, DHF), jnp.float32),
        pltpu.VMEM_SHARED((NPAD, DHF), jnp.float32),
        pltpu.SemaphoreType.DMA,
        pltpu.SemaphoreType.DMA,
        pltpu.SemaphoreType.DMA,
        pltpu.SemaphoreType.DMA,
    ],
    compiler_params=pltpu.CompilerParams(use_tc_tiling_on_sc=False),
)


BR = 1000


def _mm_body(x_ref, w_ref, o_ref):
    o_ref[...] = jnp.dot(x_ref[...], w_ref[...],
                         preferred_element_type=jnp.float32)


def _mm(x, w):
    n, d = x.shape
    return pl.pallas_call(
        _mm_body,
        grid=(n // BR,),
        in_specs=[
            pl.BlockSpec((BR, d), lambda i: (i, 0)),
            pl.BlockSpec((d, w.shape[1]), lambda i: (0, 0)),
        ],
        out_specs=pl.BlockSpec((BR, w.shape[1]), lambda i: (i, 0)),
        out_shape=jax.ShapeDtypeStruct((n, w.shape[1]), jnp.float32),
    )(x, w)


def _norm_body(d0_ref, d1_ref, xw_ref, y_ref, dinv_ref):
    deg = 1.0 + d0_ref[...] + d1_ref[...]
    dinv = lax.rsqrt(deg)
    dinv_ref[...] = dinv
    y_ref[...] = dinv * xw_ref[...]


def _norm(d0, d1, xw):
    return pl.pallas_call(
        _norm_body,
        grid=(N // BR,),
        in_specs=[
            pl.BlockSpec((BR, 1), lambda i: (i, 0)),
            pl.BlockSpec((BR, 1), lambda i: (i, 0)),
            pl.BlockSpec((BR, D_H), lambda i: (i, 0)),
        ],
        out_specs=[
            pl.BlockSpec((BR, D_H), lambda i: (i, 0)),
            pl.BlockSpec((BR, 1), lambda i: (i, 0)),
        ],
        out_shape=[
            jax.ShapeDtypeStruct((N, D_H), jnp.float32),
            jax.ShapeDtypeStruct((N, 1), jnp.float32),
        ],
    )(d0, d1, xw)


def _layer1_body(p_ref, xw_ref, dinv_ref, b_ref, h_ref, y2_ref):
    dinv = dinv_ref[...]
    agg = p_ref[0] + p_ref[1]
    pre = dinv * agg + (dinv * dinv) * xw_ref[...]
    h = jnp.maximum(pre + b_ref[...], 0.0)
    h_ref[...] = h
    y2_ref[...] = dinv * h


def _layer1(p, xw, dinv, b1):
    return pl.pallas_call(
        _layer1_body,
        grid=(N // BR,),
        in_specs=[
            pl.BlockSpec((NC, BR, D_H), lambda i: (0, i, 0)),
            pl.BlockSpec((BR, D_H), lambda i: (i, 0)),
            pl.BlockSpec((BR, 1), lambda i: (i, 0)),
            pl.BlockSpec((1, D_H), lambda i: (0, 0)),
        ],
        out_specs=[
            pl.BlockSpec((BR, D_H), lambda i: (i, 0)),
            pl.BlockSpec((BR, D_H), lambda i: (i, 0)),
        ],
        out_shape=[
            jax.ShapeDtypeStruct((N, D_H), jnp.float32),
            jax.ShapeDtypeStruct((N, D_H), jnp.float32),
        ],
    )(p, xw, dinv, b1)


def _heads_body(q_ref, h_ref, dinv_ref, wmu_ref, bmu_ref, wls_ref, bls_ref,
                mu_ref, ls_ref):
    dinv = dinv_ref[...]
    ah = dinv * (q_ref[0] + q_ref[1]) + (dinv * dinv) * h_ref[...]
    mu_ref[...] = jnp.dot(ah, wmu_ref[...],
                          preferred_element_type=jnp.float32) + bmu_ref[...]
    ls_ref[...] = jnp.dot(ah, wls_ref[...],
                          preferred_element_type=jnp.float32) + bls_ref[...]


def _heads(q, h, dinv, Wmu, bmu, Wls, bls):
    return pl.pallas_call(
        _heads_body,
        grid=(N // BR,),
        in_specs=[
            pl.BlockSpec((NC, BR, D_H), lambda i: (0, i, 0)),
            pl.BlockSpec((BR, D_H), lambda i: (i, 0)),
            pl.BlockSpec((BR, 1), lambda i: (i, 0)),
            pl.BlockSpec((D_H, D_OUT), lambda i: (0, 0)),
            pl.BlockSpec((1, D_OUT), lambda i: (0, 0)),
            pl.BlockSpec((D_H, D_OUT), lambda i: (0, 0)),
            pl.BlockSpec((1, D_OUT), lambda i: (0, 0)),
        ],
        out_specs=[
            pl.BlockSpec((BR, D_OUT), lambda i: (i, 0)),
            pl.BlockSpec((BR, D_OUT), lambda i: (i, 0)),
        ],
        out_shape=[
            jax.ShapeDtypeStruct((N, D_OUT), jnp.float32),
            jax.ShapeDtypeStruct((N, D_OUT), jnp.float32),
        ],
    )(q, h, dinv, Wmu, bmu, Wls, bls)


def kernel(x, edge_index, W1, b1, Wmu, bmu, Wls, bls):
    src = (2 * edge_index[0]).reshape(NW, K, C)
    dst = edge_index[1].reshape(NW, K, C)

    deg_parts = _deg_call(dst)
    xw = _mm(x, W1)

    d0 = lax.slice(deg_parts, (0, 0, 0), (1, N, 1)).reshape(N, 1)
    d1 = lax.slice(deg_parts, (1, 0, 0), (2, N, 1)).reshape(N, 1)
    y1, dinv = _norm(d0, d1, xw)

    p = _agg_call(y1.reshape(2 * N, DHF), src, dst)
    h, y2 = _layer1(p, xw, dinv, b1.reshape(1, D_H))

    q = _agg_call(y2.reshape(2 * N, DHF), src, dst)
    mu, ls = _heads(q, h, dinv, Wmu, bmu.reshape(1, D_OUT),
                    Wls, bls.reshape(1, D_OUT))
    return (mu, ls)


# speedup vs baseline: 39.4419x; 1.0406x over previous
import jax
import jax.numpy as jnp
from jax import lax
from jax.experimental import pallas as pl
from jax.experimental.pallas import tpu as pltpu
from jax.experimental.pallas import tpu_sc as plsc

N = 10000
NPAD = 10240
E = 320000
D_IN = 128
D_H = 128
D_OUT = 64
DHF = D_H // 2

NC = 2
NS = 16
NW = NC * NS
EPW = E // NW
C = 80
K = EPW // C
RPT = NPAD // NS

_sc_mesh = plsc.VectorSubcoreMesh(core_axis_name="c", subcore_axis_name="s")


def _deg_body(dst_hbm, out_hbm, dst_v, ones_v, zbuf, acc):
    cid = lax.axis_index("c")
    sid = lax.axis_index("s")
    wid = sid * NC + cid

    one16 = jnp.ones((16,), jnp.float32)
    zero16 = jnp.zeros((16,), jnp.float32)

    @pl.loop(0, C)
    def _(i):
        ones_v[i, :] = one16

    @pl.loop(0, RPT)
    def _(i):
        zbuf[i, :] = zero16

    pltpu.sync_copy(dst_hbm.at[wid], dst_v)
    pltpu.sync_copy(zbuf, acc.at[pl.ds(sid * RPT, RPT)])
    plsc.subcore_barrier()

    @pl.loop(0, K)
    def _(j):
        pltpu.sync_copy(ones_v, acc.at[dst_v.at[j]], add=True)

    plsc.subcore_barrier()
    pltpu.sync_copy(acc.at[pl.ds(sid * RPT, RPT)],
                    out_hbm.at[cid, pl.ds(sid * RPT, RPT)])


_deg_call = pl.kernel(
    _deg_body,
    out_type=jax.ShapeDtypeStruct((NC, NPAD, 16), jnp.float32),
    mesh=_sc_mesh,
    scratch_types=[
        pltpu.VMEM((K, C), jnp.int32),
        pltpu.VMEM((C, 16), jnp.float32),
        pltpu.VMEM((RPT, 16), jnp.float32),
        pltpu.VMEM_SHARED((NPAD, 16), jnp.float32),
    ],
    compiler_params=pltpu.CompilerParams(use_tc_tiling_on_sc=False),
)


NBUF = 4


def _agg_body(y_hbm, src_hbm, dst_hbm, out_hbm, src_v, dst_v,
              rows0, rows1, rows2, rows3, zbuf, acc,
              sem0, sem1, sem2, sem3):
    cid = lax.axis_index("c")
    sid = lax.axis_index("s")
    wid = sid * NC + cid
    rows = (rows0, rows1, rows2, rows3)
    sems = (sem0, sem1, sem2, sem3)

    zero16 = jnp.zeros((16,), jnp.float32)

    @pl.loop(0, RPT)
    def _(i):
        @pl.loop(0, DHF, step=16)
        def _(c0):
            zbuf[i, pl.ds(c0, 16)] = zero16

    pltpu.sync_copy(src_hbm.at[wid], src_v)
    pltpu.sync_copy(dst_hbm.at[wid], dst_v)

    one16i = jnp.ones((16,), jnp.int32)

    for half in range(2):
        if half == 1:
            @pl.loop(0, K)
            def _(k):
                @pl.loop(0, C, step=16)
                def _(c0):
                    src_v[k, pl.ds(c0, 16)] = src_v[k, pl.ds(c0, 16)] + one16i

        pltpu.sync_copy(zbuf, acc.at[pl.ds(sid * RPT, RPT)])
        plsc.subcore_barrier()

        y_half = y_hbm

        for b in range(NBUF):
            pltpu.async_copy(y_half.at[src_v.at[b]], rows[b], sems[b])

        @pl.loop(0, (K - NBUF) // NBUF)
        def _(i):
            j = NBUF * i
            for b in range(NBUF):
                pltpu.make_async_copy(
                    y_half.at[src_v.at[j + b]], rows[b], sems[b]).wait()
                pltpu.sync_copy(rows[b], acc.at[dst_v.at[j + b]], add=True)
                pltpu.async_copy(
                    y_half.at[src_v.at[j + b + NBUF]], rows[b], sems[b])

        base = ((K - NBUF) // NBUF) * NBUF
        for b in range(NBUF):
            pltpu.make_async_copy(
                y_half.at[src_v.at[base + b]], rows[b], sems[b]).wait()
            pltpu.sync_copy(rows[b], acc.at[dst_v.at[base + b]], add=True)
        pltpu.sync_copy(y_half.at[src_v.at[K - 1]], rows0)
        pltpu.sync_copy(rows0, acc.at[dst_v.at[K - 1]], add=True)

        plsc.subcore_barrier()
        pltpu.sync_copy(acc.at[pl.ds(sid * RPT, RPT)],
                        out_hbm.at[cid, pl.ds(sid * RPT, RPT),
                                   pl.ds(half * DHF, DHF)])


_agg_call = pl.kernel(
    _agg_body,
    out_type=jax.ShapeDtypeStruct((NC, NPAD, D_H), jnp.float32),
    mesh=_sc_mesh,
    scratch_types=[
        pltpu.VMEM((K, C), jnp.int32),
        pltpu.VMEM((K, C), jnp.int32),
        pltpu.VMEM((C, DHF), jnp.float32),
        pltpu.VMEM((C, DHF), jnp.float32),
        pltpu.VMEM((C, DHF), jnp.float32),
        pltpu.VMEM((C, DHF), jnp.float32),
        pltpu.VMEM((RPT, DHF), jnp.float32),
        pltpu.VMEM_SHARED((NPAD, DHF), jnp.float32),
        pltpu.SemaphoreType.DMA,
        pltpu.SemaphoreType.DMA,
        pltpu.SemaphoreType.DMA,
        pltpu.SemaphoreType.DMA,
    ],
    compiler_params=pltpu.CompilerParams(use_tc_tiling_on_sc=False),
)


BR = 2000


def _mm_body(x_ref, w_ref, o_ref):
    o_ref[...] = jnp.dot(x_ref[...], w_ref[...],
                         preferred_element_type=jnp.float32)


def _mm(x, w):
    n, d = x.shape
    return pl.pallas_call(
        _mm_body,
        grid=(n // BR,),
        in_specs=[
            pl.BlockSpec((BR, d), lambda i: (i, 0)),
            pl.BlockSpec((d, w.shape[1]), lambda i: (0, 0)),
        ],
        out_specs=pl.BlockSpec((BR, w.shape[1]), lambda i: (i, 0)),
        out_shape=jax.ShapeDtypeStruct((n, w.shape[1]), jnp.float32),
    )(x, w)


def _norm_body(dp_ref, xw_ref, y_ref, dinv_ref):
    deg = 1.0 + dp_ref[0, :, 0:1] + dp_ref[1, :, 0:1]
    dinv = lax.rsqrt(deg)
    dinv_ref[...] = dinv
    y_ref[...] = dinv * xw_ref[...]


def _norm(dp, xw):
    return pl.pallas_call(
        _norm_body,
        grid=(N // BR,),
        in_specs=[
            pl.BlockSpec((NC, BR, 16), lambda i: (0, i, 0)),
            pl.BlockSpec((BR, D_H), lambda i: (i, 0)),
        ],
        out_specs=[
            pl.BlockSpec((BR, D_H), lambda i: (i, 0)),
            pl.BlockSpec((BR, 1), lambda i: (i, 0)),
        ],
        out_shape=[
            jax.ShapeDtypeStruct((N, D_H), jnp.float32),
            jax.ShapeDtypeStruct((N, 1), jnp.float32),
        ],
    )(dp, xw)


def _layer1_body(p_ref, xw_ref, dinv_ref, b_ref, h_ref, y2_ref):
    dinv = dinv_ref[...]
    agg = p_ref[0] + p_ref[1]
    pre = dinv * agg + (dinv * dinv) * xw_ref[...]
    h = jnp.maximum(pre + b_ref[...], 0.0)
    h_ref[...] = h
    y2_ref[...] = dinv * h


def _layer1(p, xw, dinv, b1):
    return pl.pallas_call(
        _layer1_body,
        grid=(N // BR,),
        in_specs=[
            pl.BlockSpec((NC, BR, D_H), lambda i: (0, i, 0)),
            pl.BlockSpec((BR, D_H), lambda i: (i, 0)),
            pl.BlockSpec((BR, 1), lambda i: (i, 0)),
            pl.BlockSpec((1, D_H), lambda i: (0, 0)),
        ],
        out_specs=[
            pl.BlockSpec((BR, D_H), lambda i: (i, 0)),
            pl.BlockSpec((BR, D_H), lambda i: (i, 0)),
        ],
        out_shape=[
            jax.ShapeDtypeStruct((N, D_H), jnp.float32),
            jax.ShapeDtypeStruct((N, D_H), jnp.float32),
        ],
    )(p, xw, dinv, b1)


def _heads_body(q_ref, h_ref, dinv_ref, wmu_ref, bmu_ref, wls_ref, bls_ref,
                mu_ref, ls_ref):
    dinv = dinv_ref[...]
    ah = dinv * (q_ref[0] + q_ref[1]) + (dinv * dinv) * h_ref[...]
    mu_ref[...] = jnp.dot(ah, wmu_ref[...],
                          preferred_element_type=jnp.float32) + bmu_ref[...]
    ls_ref[...] = jnp.dot(ah, wls_ref[...],
                          preferred_element_type=jnp.float32) + bls_ref[...]


def _heads(q, h, dinv, Wmu, bmu, Wls, bls):
    return pl.pallas_call(
        _heads_body,
        grid=(N // BR,),
        in_specs=[
            pl.BlockSpec((NC, BR, D_H), lambda i: (0, i, 0)),
            pl.BlockSpec((BR, D_H), lambda i: (i, 0)),
            pl.BlockSpec((BR, 1), lambda i: (i, 0)),
            pl.BlockSpec((D_H, D_OUT), lambda i: (0, 0)),
            pl.BlockSpec((1, D_OUT), lambda i: (0, 0)),
            pl.BlockSpec((D_H, D_OUT), lambda i: (0, 0)),
            pl.BlockSpec((1, D_OUT), lambda i: (0, 0)),
        ],
        out_specs=[
            pl.BlockSpec((BR, D_OUT), lambda i: (i, 0)),
            pl.BlockSpec((BR, D_OUT), lambda i: (i, 0)),
        ],
        out_shape=[
            jax.ShapeDtypeStruct((N, D_OUT), jnp.float32),
            jax.ShapeDtypeStruct((N, D_OUT), jnp.float32),
        ],
    )(q, h, dinv, Wmu, bmu, Wls, bls)


def kernel(x, edge_index, W1, b1, Wmu, bmu, Wls, bls):
    src = (2 * edge_index[0]).reshape(NW, K, C)
    dst = edge_index[1].reshape(NW, K, C)

    deg_parts = _deg_call(dst)
    xw = _mm(x, W1)

    y1, dinv = _norm(deg_parts, xw)

    p = _agg_call(y1.reshape(2 * N, DHF), src, dst)
    h, y2 = _layer1(p, xw, dinv, b1.reshape(1, D_H))

    q = _agg_call(y2.reshape(2 * N, DHF), src, dst)
    mu, ls = _heads(q, h, dinv, Wmu, bmu.reshape(1, D_OUT),
                    Wls, bls.reshape(1, D_OUT))
    return (mu, ls)


# speedup vs baseline: 40.3754x; 1.0237x over previous
import jax
import jax.numpy as jnp
from jax import lax
from jax.experimental import pallas as pl
from jax.experimental.pallas import tpu as pltpu
from jax.experimental.pallas import tpu_sc as plsc

N = 10000
NPAD = 10240
E = 320000
D_IN = 128
D_H = 128
D_OUT = 64
DHF = D_H // 2

NC = 2
NS = 16
NW = NC * NS
EPW = E // NW
C = 80
K = EPW // C
RPT = NPAD // NS

_sc_mesh = plsc.VectorSubcoreMesh(core_axis_name="c", subcore_axis_name="s")


def _deg_body(dst_hbm, out_hbm, dst_v, ones_v, zbuf, acc):
    cid = lax.axis_index("c")
    sid = lax.axis_index("s")
    wid = sid * NC + cid

    one16 = jnp.ones((16,), jnp.float32)
    zero16 = jnp.zeros((16,), jnp.float32)

    @pl.loop(0, C)
    def _(i):
        ones_v[i, :] = one16

    @pl.loop(0, RPT)
    def _(i):
        zbuf[i, :] = zero16

    pltpu.sync_copy(dst_hbm.at[wid], dst_v)
    pltpu.sync_copy(zbuf, acc.at[pl.ds(sid * RPT, RPT)])
    plsc.subcore_barrier()

    @pl.loop(0, K)
    def _(j):
        pltpu.sync_copy(ones_v, acc.at[dst_v.at[j]], add=True)

    plsc.subcore_barrier()
    pltpu.sync_copy(acc.at[pl.ds(sid * RPT, RPT)],
                    out_hbm.at[cid, pl.ds(sid * RPT, RPT)])


_deg_call = pl.kernel(
    _deg_body,
    out_type=jax.ShapeDtypeStruct((NC, NPAD, 16), jnp.float32),
    mesh=_sc_mesh,
    scratch_types=[
        pltpu.VMEM((K, C), jnp.int32),
        pltpu.VMEM((C, 16), jnp.float32),
        pltpu.VMEM((RPT, 16), jnp.float32),
        pltpu.VMEM_SHARED((NPAD, 16), jnp.float32),
    ],
    compiler_params=pltpu.CompilerParams(use_tc_tiling_on_sc=False),
)


NBUF = 5


def _agg_body(y_hbm, src_hbm, dst_hbm, out_hbm, src_v, dst_v,
              rows0, rows1, rows2, rows3, rows4, zbuf, acc,
              sem0, sem1, sem2, sem3, sem4):
    cid = lax.axis_index("c")
    sid = lax.axis_index("s")
    wid = sid * NC + cid
    rows = (rows0, rows1, rows2, rows3, rows4)
    sems = (sem0, sem1, sem2, sem3, sem4)

    zero16 = jnp.zeros((16,), jnp.float32)

    @pl.loop(0, RPT)
    def _(i):
        @pl.loop(0, DHF, step=16)
        def _(c0):
            zbuf[i, pl.ds(c0, 16)] = zero16

    pltpu.sync_copy(src_hbm.at[wid], src_v)
    pltpu.sync_copy(dst_hbm.at[wid], dst_v)

    one16i = jnp.ones((16,), jnp.int32)

    for half in range(2):
        if half == 1:
            @pl.loop(0, K)
            def _(k):
                @pl.loop(0, C, step=16)
                def _(c0):
                    src_v[k, pl.ds(c0, 16)] = src_v[k, pl.ds(c0, 16)] + one16i

        pltpu.sync_copy(zbuf, acc.at[pl.ds(sid * RPT, RPT)])
        plsc.subcore_barrier()

        y_half = y_hbm

        for b in range(NBUF):
            pltpu.async_copy(y_half.at[src_v.at[b]], rows[b], sems[b])

        @pl.loop(0, (K - NBUF) // NBUF)
        def _(i):
            j = NBUF * i
            for b in range(NBUF):
                pltpu.make_async_copy(
                    y_half.at[src_v.at[j + b]], rows[b], sems[b]).wait()
                pltpu.sync_copy(rows[b], acc.at[dst_v.at[j + b]], add=True)
                pltpu.async_copy(
                    y_half.at[src_v.at[j + b + NBUF]], rows[b], sems[b])

        base = ((K - NBUF) // NBUF) * NBUF
        for b in range(NBUF):
            pltpu.make_async_copy(
                y_half.at[src_v.at[base + b]], rows[b], sems[b]).wait()
            pltpu.sync_copy(rows[b], acc.at[dst_v.at[base + b]], add=True)

        plsc.subcore_barrier()
        pltpu.sync_copy(acc.at[pl.ds(sid * RPT, RPT)],
                        out_hbm.at[cid, pl.ds(sid * RPT, RPT),
                                   pl.ds(half * DHF, DHF)])


_agg_call = pl.kernel(
    _agg_body,
    out_type=jax.ShapeDtypeStruct((NC, NPAD, D_H), jnp.float32),
    mesh=_sc_mesh,
    scratch_types=[
        pltpu.VMEM((K, C), jnp.int32),
        pltpu.VMEM((K, C), jnp.int32),
        pltpu.VMEM((C, DHF), jnp.float32),
        pltpu.VMEM((C, DHF), jnp.float32),
        pltpu.VMEM((C, DHF), jnp.float32),
        pltpu.VMEM((C, DHF), jnp.float32),
        pltpu.VMEM((C, DHF), jnp.float32),
        pltpu.VMEM((RPT, DHF), jnp.float32),
        pltpu.VMEM_SHARED((NPAD, DHF), jnp.float32),
        pltpu.SemaphoreType.DMA,
        pltpu.SemaphoreType.DMA,
        pltpu.SemaphoreType.DMA,
        pltpu.SemaphoreType.DMA,
        pltpu.SemaphoreType.DMA,
    ],
    compiler_params=pltpu.CompilerParams(use_tc_tiling_on_sc=False),
)


BR = 2000


def _mm_body(x_ref, w_ref, o_ref):
    o_ref[...] = jnp.dot(x_ref[...], w_ref[...],
                         preferred_element_type=jnp.float32)


def _mm(x, w):
    n, d = x.shape
    return pl.pallas_call(
        _mm_body,
        grid=(n // BR,),
        in_specs=[
            pl.BlockSpec((BR, d), lambda i: (i, 0)),
            pl.BlockSpec((d, w.shape[1]), lambda i: (0, 0)),
        ],
        out_specs=pl.BlockSpec((BR, w.shape[1]), lambda i: (i, 0)),
        out_shape=jax.ShapeDtypeStruct((n, w.shape[1]), jnp.float32),
    )(x, w)


def _norm_body(dp_ref, xw_ref, y_ref, dinv_ref):
    deg = 1.0 + dp_ref[0, :, 0:1] + dp_ref[1, :, 0:1]
    dinv = lax.rsqrt(deg)
    dinv_ref[...] = dinv
    y_ref[...] = dinv * xw_ref[...]


def _norm(dp, xw):
    return pl.pallas_call(
        _norm_body,
        grid=(N // BR,),
        in_specs=[
            pl.BlockSpec((NC, BR, 16), lambda i: (0, i, 0)),
            pl.BlockSpec((BR, D_H), lambda i: (i, 0)),
        ],
        out_specs=[
            pl.BlockSpec((BR, D_H), lambda i: (i, 0)),
            pl.BlockSpec((BR, 1), lambda i: (i, 0)),
        ],
        out_shape=[
            jax.ShapeDtypeStruct((N, D_H), jnp.float32),
            jax.ShapeDtypeStruct((N, 1), jnp.float32),
        ],
    )(dp, xw)


def _layer1_body(p_ref, xw_ref, dinv_ref, b_ref, h_ref, y2_ref):
    dinv = dinv_ref[...]
    agg = p_ref[0] + p_ref[1]
    pre = dinv * agg + (dinv * dinv) * xw_ref[...]
    h = jnp.maximum(pre + b_ref[...], 0.0)
    h_ref[...] = h
    y2_ref[...] = dinv * h


def _layer1(p, xw, dinv, b1):
    return pl.pallas_call(
        _layer1_body,
        grid=(N // BR,),
        in_specs=[
            pl.BlockSpec((NC, BR, D_H), lambda i: (0, i, 0)),
            pl.BlockSpec((BR, D_H), lambda i: (i, 0)),
            pl.BlockSpec((BR, 1), lambda i: (i, 0)),
            pl.BlockSpec((1, D_H), lambda i: (0, 0)),
        ],
        out_specs=[
            pl.BlockSpec((BR, D_H), lambda i: (i, 0)),
            pl.BlockSpec((BR, D_H), lambda i: (i, 0)),
        ],
        out_shape=[
            jax.ShapeDtypeStruct((N, D_H), jnp.float32),
            jax.ShapeDtypeStruct((N, D_H), jnp.float32),
        ],
    )(p, xw, dinv, b1)


def _heads_body(q_ref, h_ref, dinv_ref, wmu_ref, bmu_ref, wls_ref, bls_ref,
                mu_ref, ls_ref):
    dinv = dinv_ref[...]
    ah = dinv * (q_ref[0] + q_ref[1]) + (dinv * dinv) * h_ref[...]
    mu_ref[...] = jnp.dot(ah, wmu_ref[...],
                          preferred_element_type=jnp.float32) + bmu_ref[...]
    ls_ref[...] = jnp.dot(ah, wls_ref[...],
                          preferred_element_type=jnp.float32) + bls_ref[...]


def _heads(q, h, dinv, Wmu, bmu, Wls, bls):
    return pl.pallas_call(
        _heads_body,
        grid=(N // BR,),
        in_specs=[
            pl.BlockSpec((NC, BR, D_H), lambda i: (0, i, 0)),
            pl.BlockSpec((BR, D_H), lambda i: (i, 0)),
            pl.BlockSpec((BR, 1), lambda i: (i, 0)),
            pl.BlockSpec((D_H, D_OUT), lambda i: (0, 0)),
            pl.BlockSpec((1, D_OUT), lambda i: (0, 0)),
            pl.BlockSpec((D_H, D_OUT), lambda i: (0, 0)),
            pl.BlockSpec((1, D_OUT), lambda i: (0, 0)),
        ],
        out_specs=[
            pl.BlockSpec((BR, D_OUT), lambda i: (i, 0)),
            pl.BlockSpec((BR, D_OUT), lambda i: (i, 0)),
        ],
        out_shape=[
            jax.ShapeDtypeStruct((N, D_OUT), jnp.float32),
            jax.ShapeDtypeStruct((N, D_OUT), jnp.float32),
        ],
    )(q, h, dinv, Wmu, bmu, Wls, bls)


def kernel(x, edge_index, W1, b1, Wmu, bmu, Wls, bls):
    src = (2 * edge_index[0]).reshape(NW, K, C)
    dst = edge_index[1].reshape(NW, K, C)

    deg_parts = _deg_call(dst)
    xw = _mm(x, W1)

    y1, dinv = _norm(deg_parts, xw)

    p = _agg_call(y1.reshape(2 * N, DHF), src, dst)
    h, y2 = _layer1(p, xw, dinv, b1.reshape(1, D_H))

    q = _agg_call(y2.reshape(2 * N, DHF), src, dst)
    mu, ls = _heads(q, h, dinv, Wmu, bmu.reshape(1, D_OUT),
                    Wls, bls.reshape(1, D_OUT))
    return (mu, ls)
